# baseline (speedup 1.0000x reference)
import functools

import jax
import jax.numpy as jnp
from jax.experimental import pallas as pl
from jax.experimental.pallas import tpu as pltpu

_Q = 900
_QPAD = 1024
_C = 91
_CPAD = 128
_K = 300
_KPAD = 512
_TOK = 384
_IOU_T = 0.7
_BIG = 1 << 30


def _body(logits_ref, pm_ref, pb_ref, scale_ref,
          scores_ref, labels_ref, boxes_ref, keep_ref,
          prob_ref, rowmax_ref, bxt_ref, boxsel_ref,
          scorerow_ref, labelrow_ref, keeprow_ref, iou_ref,
          r_x0, r_y0, r_x1, r_y1, c_x0, c_y0, c_x1, c_y1):
    f32 = jnp.float32

    x = logits_ref[0]
    sig = jax.nn.sigmoid(x)
    pm = pm_ref[...]
    rs = jnp.sum(pm, axis=1, keepdims=True)
    w = jnp.where(rs > 0, pm / jnp.maximum(rs, 1e-6), pm)
    prob = jax.lax.dot_general(sig, w, (((1,), (1,)), ((), ())),
                               preferred_element_type=f32)
    prob_ref[...] = jnp.full((_QPAD, _CPAD), -1.0, f32)
    prob_ref[0:_Q, 0:_C] = prob
    rowmax_ref[...] = jnp.max(prob_ref[...], axis=1, keepdims=True)

    pb = pb_ref[0]
    img_w = scale_ref[0, 0:1, 0:1]
    img_h = scale_ref[0, 0:1, 1:2]
    cx, cy = pb[:, 0:1], pb[:, 1:2]
    bw, bh = pb[:, 2:3], pb[:, 3:4]
    bxt_ref[0:_Q, 0:1] = (cx - 0.5 * bw) * img_w
    bxt_ref[0:_Q, 1:2] = (cy - 0.5 * bh) * img_h
    bxt_ref[0:_Q, 2:3] = (cx + 0.5 * bw) * img_w
    bxt_ref[0:_Q, 3:4] = (cy + 0.5 * bh) * img_h

    sub_q = jax.lax.broadcasted_iota(jnp.int32, (_QPAD, 1), 0)
    lane_c = jax.lax.broadcasted_iota(jnp.int32, (1, _CPAD), 1)
    lane_k = jax.lax.broadcasted_iota(jnp.int32, (1, _KPAD), 1)
    sub_k = jax.lax.broadcasted_iota(jnp.int32, (_KPAD, 1), 0)

    zrow = jnp.zeros((1, _KPAD), f32)
    scorerow_ref[...] = zrow
    labelrow_ref[...] = jnp.zeros((1, _KPAD), jnp.int32)
    boxsel_ref[...] = jnp.zeros((_KPAD, 4), f32)
    for ref in (r_x0, r_y0, r_x1, r_y1):
        ref[...] = zrow
    zcol = jnp.zeros((_KPAD, 1), f32)
    for ref in (c_x0, c_y0, c_x1, c_y1):
        ref[...] = zcol

    def extract(i, carry):
        rm = rowmax_ref[...]
        m = jnp.max(rm)
        rid = jnp.min(jnp.where(rm == m, sub_q, _BIG))
        row = prob_ref[pl.ds(rid, 1), :]
        cid = jnp.min(jnp.where(row == m, lane_c, _BIG))
        nrow = jnp.where(lane_c == cid, -1.0, row)
        prob_ref[pl.ds(rid, 1), :] = nrow
        rowmax_ref[...] = jnp.where(sub_q == rid, jnp.max(nrow), rm)

        scorerow_ref[...] = jnp.where(lane_k == i, m, scorerow_ref[...])
        labelrow_ref[...] = jnp.where(lane_k == i, cid, labelrow_ref[...])

        brow = bxt_ref[pl.ds(rid, 1), 0:4]
        boxsel_ref[...] = jnp.where(sub_k == i, brow, boxsel_ref[...])
        for j, (rref, cref) in enumerate(
                ((r_x0, c_x0), (r_y0, c_y0), (r_x1, c_x1), (r_y1, c_y1))):
            v = brow[0:1, j:j + 1]
            rref[...] = jnp.where(lane_k == i, v, rref[...])
            cref[...] = jnp.where(sub_k == i, v, cref[...])
        return carry

    jax.lax.fori_loop(0, _K, extract, 0)

    x0r, y0r, x1r, y1r = r_x0[...], r_y0[...], r_x1[...], r_y1[...]
    x0c, y0c, x1c, y1c = c_x0[...], c_y0[...], c_x1[...], c_y1[...]
    area_r = jnp.maximum(x1r - x0r, 0.0) * jnp.maximum(y1r - y0r, 0.0)
    area_c = jnp.maximum(x1c - x0c, 0.0) * jnp.maximum(y1c - y0c, 0.0)
    iw = jnp.maximum(jnp.minimum(x1c, x1r) - jnp.maximum(x0c, x0r), 0.0)
    ih = jnp.maximum(jnp.minimum(y1c, y1r) - jnp.maximum(y0c, y0r), 0.0)
    inter = iw * ih
    union = area_c + area_r - inter
    iou_ref[...] = inter / jnp.maximum(union, 1e-9)

    keeprow_ref[...] = jnp.ones((1, _KPAD), f32)

    def nms(i, carry):
        krow = keeprow_ref[...]
        ki = jnp.max(jnp.where(lane_k == i, krow, 0.0))
        iourow = iou_ref[pl.ds(i, 1), :]
        sup = (iourow > _IOU_T) & (lane_k > i) & (ki > 0.5)
        keeprow_ref[...] = jnp.where(sup, 0.0, krow)
        return carry

    jax.lax.fori_loop(0, _K, nms, 0)

    scores_ref[0] = scorerow_ref[...]
    labels_ref[0] = labelrow_ref[...]
    keep_ref[0] = keeprow_ref[...]
    boxes_ref[...] = boxsel_ref[...].reshape(1, _KPAD, 4)


@functools.partial(jax.jit)
def _run(lsl, pm, pb, scale):
    f32 = jnp.float32
    grid = (lsl.shape[0],)
    out = pl.pallas_call(
        _body,
        grid=grid,
        in_specs=[
            pl.BlockSpec((1, _Q, _TOK), lambda b: (b, 0, 0)),
            pl.BlockSpec((_C, _TOK), lambda b: (0, 0)),
            pl.BlockSpec((1, _Q, 4), lambda b: (b, 0, 0)),
            pl.BlockSpec((1, 1, 4), lambda b: (b, 0, 0)),
        ],
        out_specs=[
            pl.BlockSpec((1, 1, _KPAD), lambda b: (b, 0, 0)),
            pl.BlockSpec((1, 1, _KPAD), lambda b: (b, 0, 0)),
            pl.BlockSpec((1, _KPAD, 4), lambda b: (b, 0, 0)),
            pl.BlockSpec((1, 1, _KPAD), lambda b: (b, 0, 0)),
        ],
        out_shape=[
            jax.ShapeDtypeStruct((grid[0], 1, _KPAD), f32),
            jax.ShapeDtypeStruct((grid[0], 1, _KPAD), jnp.int32),
            jax.ShapeDtypeStruct((grid[0], _KPAD, 4), f32),
            jax.ShapeDtypeStruct((grid[0], 1, _KPAD), f32),
        ],
        scratch_shapes=[
            pltpu.VMEM((_QPAD, _CPAD), f32),
            pltpu.VMEM((_QPAD, 1), f32),
            pltpu.VMEM((_QPAD, 4), f32),
            pltpu.VMEM((_KPAD, 4), f32),
            pltpu.VMEM((1, _KPAD), f32),
            pltpu.VMEM((1, _KPAD), jnp.int32),
            pltpu.VMEM((1, _KPAD), f32),
            pltpu.VMEM((_KPAD, _KPAD), f32),
            pltpu.VMEM((1, _KPAD), f32), pltpu.VMEM((1, _KPAD), f32),
            pltpu.VMEM((1, _KPAD), f32), pltpu.VMEM((1, _KPAD), f32),
            pltpu.VMEM((_KPAD, 1), f32), pltpu.VMEM((_KPAD, 1), f32),
            pltpu.VMEM((_KPAD, 1), f32), pltpu.VMEM((_KPAD, 1), f32),
        ],
        compiler_params=pltpu.CompilerParams(
            dimension_semantics=("parallel",)),
    )(lsl, pm, pb, scale)
    return out


def kernel(pred_logits, pred_boxes, pos_map, target_sizes):
    lsl = pred_logits[:, :, :_TOK]
    pm = pos_map[:, :_TOK]
    img_h = target_sizes[:, 0].astype(jnp.float32)
    img_w = target_sizes[:, 1].astype(jnp.float32)
    scale = jnp.stack([img_w, img_h, img_w, img_h], axis=1)[:, None, :]
    s, l, bx, k = _run(lsl, pm, pred_boxes, scale)
    scores = s[:, 0, :_K]
    labels = l[:, 0, :_K]
    boxes = bx[:, :_K, :]
    keep = k[:, 0, :_K] > 0.5
    return scores, labels, boxes, keep


# speedup vs baseline: 1.5462x; 1.5462x over previous
import functools

import jax
import jax.numpy as jnp
from jax.experimental import pallas as pl
from jax.experimental.pallas import tpu as pltpu

_Q = 900
_QPAD = 1024
_C = 91
_CPAD = 128
_K = 300
_KPAD = 512
_TOK = 384
_IOU_T = 0.7
_BIG = 1 << 30


def _body(logits_ref, pm_ref, pb_ref, scale_ref,
          scores_ref, labels_ref, boxes_ref, keep_ref,
          prob_ref, rowmax_ref, bxt_ref,
          scorerow_ref, labelrow_ref, keeprow_ref, iou_ref,
          r_x0, r_y0, r_x1, r_y1):
    f32 = jnp.float32

    x = logits_ref[0]
    sig = jax.nn.sigmoid(x)
    pm = pm_ref[...]
    rs = jnp.sum(pm, axis=1, keepdims=True)
    w = jnp.where(rs > 0, pm / jnp.maximum(rs, 1e-6), pm)
    prob = jax.lax.dot_general(sig, w, (((1,), (1,)), ((), ())),
                               preferred_element_type=f32)
    prob_ref[...] = jnp.full((_QPAD, _CPAD), -1.0, f32)
    prob_ref[0:_Q, 0:_C] = prob
    pf = prob_ref[...]
    rowmax_ref[...] = jnp.max(pf.reshape(8, 128, _CPAD), axis=2)

    pb = pb_ref[0]
    img_w = scale_ref[0, 0:1, 0:1]
    img_h = scale_ref[0, 0:1, 1:2]
    cx, cy = pb[:, 0:1], pb[:, 1:2]
    bw, bh = pb[:, 2:3], pb[:, 3:4]
    bxt_ref[0:_Q, 0:1] = (cx - 0.5 * bw) * img_w
    bxt_ref[0:_Q, 1:2] = (cy - 0.5 * bh) * img_h
    bxt_ref[0:_Q, 2:3] = (cx + 0.5 * bw) * img_w
    bxt_ref[0:_Q, 3:4] = (cy + 0.5 * bh) * img_h

    flat_rq = (jax.lax.broadcasted_iota(jnp.int32, (8, 128), 0) * 128
               + jax.lax.broadcasted_iota(jnp.int32, (8, 128), 1))
    lane_c = jax.lax.broadcasted_iota(jnp.int32, (1, _CPAD), 1)
    lane_k = jax.lax.broadcasted_iota(jnp.int32, (1, _KPAD), 1)

    zrow = jnp.zeros((1, _KPAD), f32)
    scorerow_ref[...] = zrow
    labelrow_ref[...] = jnp.zeros((1, _KPAD), jnp.int32)
    for ref in (r_x0, r_y0, r_x1, r_y1):
        ref[...] = zrow

    def extract(i, carry):
        rm = rowmax_ref[...]
        m = jnp.max(rm)
        rid = jnp.min(jnp.where(rm == m, flat_rq, _BIG))
        row = prob_ref[pl.ds(rid, 1), :]
        cid = jnp.min(jnp.where(row == m, lane_c, _BIG))
        nrow = jnp.where(lane_c == cid, -1.0, row)
        prob_ref[pl.ds(rid, 1), :] = nrow
        rowmax_ref[...] = jnp.where(flat_rq == rid, jnp.max(nrow), rm)

        sel = lane_k == i
        scorerow_ref[...] = jnp.where(sel, m, scorerow_ref[...])
        labelrow_ref[...] = jnp.where(sel, cid, labelrow_ref[...])

        brow = bxt_ref[pl.ds(rid, 1), 0:4]
        r_x0[...] = jnp.where(sel, brow[0:1, 0:1], r_x0[...])
        r_y0[...] = jnp.where(sel, brow[0:1, 1:2], r_y0[...])
        r_x1[...] = jnp.where(sel, brow[0:1, 2:3], r_x1[...])
        r_y1[...] = jnp.where(sel, brow[0:1, 3:4], r_y1[...])
        return carry

    jax.lax.fori_loop(0, _K, extract, 0)

    x0r, y0r, x1r, y1r = r_x0[...], r_y0[...], r_x1[...], r_y1[...]
    x0c = jnp.transpose(x0r)
    y0c = jnp.transpose(y0r)
    x1c = jnp.transpose(x1r)
    y1c = jnp.transpose(y1r)
    area_r = jnp.maximum(x1r - x0r, 0.0) * jnp.maximum(y1r - y0r, 0.0)
    area_c = jnp.maximum(x1c - x0c, 0.0) * jnp.maximum(y1c - y0c, 0.0)
    iw = jnp.maximum(jnp.minimum(x1c, x1r) - jnp.maximum(x0c, x0r), 0.0)
    ih = jnp.maximum(jnp.minimum(y1c, y1r) - jnp.maximum(y0c, y0r), 0.0)
    inter = iw * ih
    union = area_c + area_r - inter
    iou_ref[...] = inter / jnp.maximum(union, 1e-9)

    keeprow_ref[...] = jnp.ones((1, _KPAD), f32)

    def nms(i, carry):
        krow = keeprow_ref[...]
        ki = jnp.max(jnp.where(lane_k == i, krow, 0.0))
        iourow = iou_ref[pl.ds(i, 1), :]
        sup = (iourow > _IOU_T) & (lane_k > i) & (ki > 0.5)
        keeprow_ref[...] = jnp.where(sup, 0.0, krow)
        return carry

    jax.lax.fori_loop(0, _K, nms, 0)

    scores_ref[0] = scorerow_ref[...]
    labels_ref[0] = labelrow_ref[...]
    keep_ref[0] = keeprow_ref[...]
    boxes_ref[0, :, 0:1] = x0c
    boxes_ref[0, :, 1:2] = y0c
    boxes_ref[0, :, 2:3] = x1c
    boxes_ref[0, :, 3:4] = y1c


@functools.partial(jax.jit)
def _run(lsl, pm, pb, scale):
    f32 = jnp.float32
    grid = (lsl.shape[0],)
    out = pl.pallas_call(
        _body,
        grid=grid,
        in_specs=[
            pl.BlockSpec((1, _Q, _TOK), lambda b: (b, 0, 0)),
            pl.BlockSpec((_C, _TOK), lambda b: (0, 0)),
            pl.BlockSpec((1, _Q, 4), lambda b: (b, 0, 0)),
            pl.BlockSpec((1, 1, 4), lambda b: (b, 0, 0)),
        ],
        out_specs=[
            pl.BlockSpec((1, 1, _KPAD), lambda b: (b, 0, 0)),
            pl.BlockSpec((1, 1, _KPAD), lambda b: (b, 0, 0)),
            pl.BlockSpec((1, _KPAD, 4), lambda b: (b, 0, 0)),
            pl.BlockSpec((1, 1, _KPAD), lambda b: (b, 0, 0)),
        ],
        out_shape=[
            jax.ShapeDtypeStruct((grid[0], 1, _KPAD), f32),
            jax.ShapeDtypeStruct((grid[0], 1, _KPAD), jnp.int32),
            jax.ShapeDtypeStruct((grid[0], _KPAD, 4), f32),
            jax.ShapeDtypeStruct((grid[0], 1, _KPAD), f32),
        ],
        scratch_shapes=[
            pltpu.VMEM((_QPAD, _CPAD), f32),
            pltpu.VMEM((8, 128), f32),
            pltpu.VMEM((_QPAD, 4), f32),
            pltpu.VMEM((1, _KPAD), f32),
            pltpu.VMEM((1, _KPAD), jnp.int32),
            pltpu.VMEM((1, _KPAD), f32),
            pltpu.VMEM((_KPAD, _KPAD), f32),
            pltpu.VMEM((1, _KPAD), f32), pltpu.VMEM((1, _KPAD), f32),
            pltpu.VMEM((1, _KPAD), f32), pltpu.VMEM((1, _KPAD), f32),
        ],
        compiler_params=pltpu.CompilerParams(
            dimension_semantics=("parallel",)),
    )(lsl, pm, pb, scale)
    return out


def kernel(pred_logits, pred_boxes, pos_map, target_sizes):
    lsl = pred_logits[:, :, :_TOK]
    pm = pos_map[:, :_TOK]
    img_h = target_sizes[:, 0].astype(jnp.float32)
    img_w = target_sizes[:, 1].astype(jnp.float32)
    scale = jnp.stack([img_w, img_h, img_w, img_h], axis=1)[:, None, :]
    s, l, bx, k = _run(lsl, pm, pred_boxes, scale)
    scores = s[:, 0, :_K]
    labels = l[:, 0, :_K]
    boxes = bx[:, :_K, :]
    keep = k[:, 0, :_K] > 0.5
    return scores, labels, boxes, keep


# speedup vs baseline: 1.5621x; 1.0103x over previous
import functools

import jax
import jax.numpy as jnp
from jax.experimental import pallas as pl
from jax.experimental.pallas import tpu as pltpu

_Q = 900
_QPAD = 1024
_C = 91
_CPAD = 128
_K = 300
_KPAD = 512
_TOK = 384
_IOU_T = 0.7
_BIG = 1 << 30


def _body(logits_ref, pm_ref, pb_ref, scale_ref,
          scores_ref, labels_ref, boxes_ref, keep_ref,
          prob_ref, rowmax_ref, bxt_ref,
          scorerow_ref, labelrow_ref, ridrow_ref, keeprow_ref, iou_ref):
    f32 = jnp.float32

    x = logits_ref[0]
    sig = jax.nn.sigmoid(x)
    pm = pm_ref[...]
    rs = jnp.sum(pm, axis=1, keepdims=True)
    w = jnp.where(rs > 0, pm / jnp.maximum(rs, 1e-6), pm)
    prob = jax.lax.dot_general(sig, w, (((1,), (1,)), ((), ())),
                               preferred_element_type=f32)
    prob_ref[...] = jnp.full((_QPAD, _CPAD), -1.0, f32)
    prob_ref[0:_Q, 0:_C] = prob
    pf = prob_ref[...]
    rowmax_ref[...] = jnp.max(pf.reshape(8, 128, _CPAD), axis=2)

    bxt_ref[...] = jnp.zeros((_QPAD, 4), f32)
    pb = pb_ref[0]
    img_w = scale_ref[0, 0:1, 0:1]
    img_h = scale_ref[0, 0:1, 1:2]
    cx, cy = pb[:, 0:1], pb[:, 1:2]
    bw, bh = pb[:, 2:3], pb[:, 3:4]
    bxt_ref[0:_Q, 0:1] = (cx - 0.5 * bw) * img_w
    bxt_ref[0:_Q, 1:2] = (cy - 0.5 * bh) * img_h
    bxt_ref[0:_Q, 2:3] = (cx + 0.5 * bw) * img_w
    bxt_ref[0:_Q, 3:4] = (cy + 0.5 * bh) * img_h

    flat_rq = (jax.lax.broadcasted_iota(jnp.int32, (8, 128), 0) * 128
               + jax.lax.broadcasted_iota(jnp.int32, (8, 128), 1))
    lane_c = jax.lax.broadcasted_iota(jnp.int32, (1, _CPAD), 1)
    lane_k = jax.lax.broadcasted_iota(jnp.int32, (1, _KPAD), 1)

    scorerow_ref[...] = jnp.zeros((1, _KPAD), f32)
    labelrow_ref[...] = jnp.zeros((1, _KPAD), jnp.int32)
    ridrow_ref[...] = jnp.zeros((1, _KPAD), jnp.int32)

    def extract(i, carry):
        rm = rowmax_ref[...]
        m = jnp.max(rm)
        rid = jnp.min(jnp.where(rm == m, flat_rq, _BIG))
        row = prob_ref[pl.ds(rid, 1), :]
        cid = jnp.min(jnp.where(row == m, lane_c, _BIG))
        nrow = jnp.where(lane_c == cid, -1.0, row)
        prob_ref[pl.ds(rid, 1), :] = nrow
        rowmax_ref[...] = jnp.where(flat_rq == rid, jnp.max(nrow), rm)

        sel = lane_k == i
        scorerow_ref[...] = jnp.where(sel, m, scorerow_ref[...])
        labelrow_ref[...] = jnp.where(sel, cid, labelrow_ref[...])
        ridrow_ref[...] = jnp.where(sel, rid, ridrow_ref[...])
        return carry

    jax.lax.fori_loop(0, _K, extract, 0)

    rid_c = jnp.transpose(ridrow_ref[...])
    onehot = (rid_c == jax.lax.broadcasted_iota(
        jnp.int32, (1, _QPAD), 1)).astype(f32)
    bsel = jax.lax.dot_general(
        onehot, bxt_ref[...], (((1,), (0,)), ((), ())),
        precision=jax.lax.Precision.HIGHEST,
        preferred_element_type=f32)
    x0c = bsel[:, 0:1]
    y0c = bsel[:, 1:2]
    x1c = bsel[:, 2:3]
    y1c = bsel[:, 3:4]
    x0r = jnp.transpose(x0c)
    y0r = jnp.transpose(y0c)
    x1r = jnp.transpose(x1c)
    y1r = jnp.transpose(y1c)
    area_r = jnp.maximum(x1r - x0r, 0.0) * jnp.maximum(y1r - y0r, 0.0)
    area_c = jnp.maximum(x1c - x0c, 0.0) * jnp.maximum(y1c - y0c, 0.0)
    iw = jnp.maximum(jnp.minimum(x1c, x1r) - jnp.maximum(x0c, x0r), 0.0)
    ih = jnp.maximum(jnp.minimum(y1c, y1r) - jnp.maximum(y0c, y0r), 0.0)
    inter = iw * ih
    union = area_c + area_r - inter
    iou_ref[...] = inter / jnp.maximum(union, 1e-9)

    keeprow_ref[...] = jnp.ones((1, _KPAD), f32)

    def nms(i, carry):
        krow = keeprow_ref[...]
        ki = jnp.max(jnp.where(lane_k == i, krow, 0.0))
        iourow = iou_ref[pl.ds(i, 1), :]
        sup = (iourow > _IOU_T) & (lane_k > i) & (ki > 0.5)
        keeprow_ref[...] = jnp.where(sup, 0.0, krow)
        return carry

    jax.lax.fori_loop(0, _K, nms, 0)

    scores_ref[0] = scorerow_ref[...]
    labels_ref[0] = labelrow_ref[...]
    keep_ref[0] = keeprow_ref[...]
    boxes_ref[0, :, 0:1] = x0c
    boxes_ref[0, :, 1:2] = y0c
    boxes_ref[0, :, 2:3] = x1c
    boxes_ref[0, :, 3:4] = y1c


@functools.partial(jax.jit)
def _run(lsl, pm, pb, scale):
    f32 = jnp.float32
    grid = (lsl.shape[0],)
    out = pl.pallas_call(
        _body,
        grid=grid,
        in_specs=[
            pl.BlockSpec((1, _Q, _TOK), lambda b: (b, 0, 0)),
            pl.BlockSpec((_C, _TOK), lambda b: (0, 0)),
            pl.BlockSpec((1, _Q, 4), lambda b: (b, 0, 0)),
            pl.BlockSpec((1, 1, 4), lambda b: (b, 0, 0)),
        ],
        out_specs=[
            pl.BlockSpec((1, 1, _KPAD), lambda b: (b, 0, 0)),
            pl.BlockSpec((1, 1, _KPAD), lambda b: (b, 0, 0)),
            pl.BlockSpec((1, _KPAD, 4), lambda b: (b, 0, 0)),
            pl.BlockSpec((1, 1, _KPAD), lambda b: (b, 0, 0)),
        ],
        out_shape=[
            jax.ShapeDtypeStruct((grid[0], 1, _KPAD), f32),
            jax.ShapeDtypeStruct((grid[0], 1, _KPAD), jnp.int32),
            jax.ShapeDtypeStruct((grid[0], _KPAD, 4), f32),
            jax.ShapeDtypeStruct((grid[0], 1, _KPAD), f32),
        ],
        scratch_shapes=[
            pltpu.VMEM((_QPAD, _CPAD), f32),
            pltpu.VMEM((8, 128), f32),
            pltpu.VMEM((_QPAD, 4), f32),
            pltpu.VMEM((1, _KPAD), f32),
            pltpu.VMEM((1, _KPAD), jnp.int32),
            pltpu.VMEM((1, _KPAD), jnp.int32),
            pltpu.VMEM((1, _KPAD), f32),
            pltpu.VMEM((_KPAD, _KPAD), f32),
        ],
        compiler_params=pltpu.CompilerParams(
            dimension_semantics=("parallel",)),
    )(lsl, pm, pb, scale)
    return out


def kernel(pred_logits, pred_boxes, pos_map, target_sizes):
    lsl = pred_logits[:, :, :_TOK]
    pm = pos_map[:, :_TOK]
    img_h = target_sizes[:, 0].astype(jnp.float32)
    img_w = target_sizes[:, 1].astype(jnp.float32)
    scale = jnp.stack([img_w, img_h, img_w, img_h], axis=1)[:, None, :]
    s, l, bx, k = _run(lsl, pm, pred_boxes, scale)
    scores = s[:, 0, :_K]
    labels = l[:, 0, :_K]
    boxes = bx[:, :_K, :]
    keep = k[:, 0, :_K] > 0.5
    return scores, labels, boxes, keep


# speedup vs baseline: 1.6488x; 1.0555x over previous
import functools

import jax
import jax.numpy as jnp
from jax.experimental import pallas as pl
from jax.experimental.pallas import tpu as pltpu

_Q = 900
_QPAD = 1024
_C = 91
_CPAD = 128
_K = 300
_KPAD = 512
_TOK = 384
_IOU_T = 0.7
_BIG = 1 << 30


def _body(logits_ref, pm_ref, pb_ref, scale_ref,
          scores_ref, labels_ref, boxes_ref, keep_ref,
          prob_ref, bxt_ref,
          scorerow_ref, labelrow_ref, ridrow_ref, keeprow_ref, iou_ref):
    f32 = jnp.float32

    x = logits_ref[0]
    sig = jax.nn.sigmoid(x)
    pm = pm_ref[...]
    rs = jnp.sum(pm, axis=1, keepdims=True)
    w = jnp.where(rs > 0, pm / jnp.maximum(rs, 1e-6), pm)
    prob = jax.lax.dot_general(sig, w, (((1,), (1,)), ((), ())),
                               preferred_element_type=f32)
    prob_ref[...] = jnp.full((_QPAD, _CPAD), -1.0, f32)
    prob_ref[0:_Q, 0:_C] = prob

    bxt_ref[...] = jnp.zeros((_QPAD, 4), f32)
    pb = pb_ref[0]
    img_w = scale_ref[0, 0:1, 0:1]
    img_h = scale_ref[0, 0:1, 1:2]
    cx, cy = pb[:, 0:1], pb[:, 1:2]
    bw, bh = pb[:, 2:3], pb[:, 3:4]
    bxt_ref[0:_Q, 0:1] = (cx - 0.5 * bw) * img_w
    bxt_ref[0:_Q, 1:2] = (cy - 0.5 * bh) * img_h
    bxt_ref[0:_Q, 2:3] = (cx + 0.5 * bw) * img_w
    bxt_ref[0:_Q, 3:4] = (cy + 0.5 * bh) * img_h

    flat2d = (jax.lax.broadcasted_iota(jnp.int32, (_QPAD, _CPAD), 0) * _CPAD
              + jax.lax.broadcasted_iota(jnp.int32, (_QPAD, _CPAD), 1))
    lane_k = jax.lax.broadcasted_iota(jnp.int32, (1, _KPAD), 1)

    scorerow_ref[...] = jnp.zeros((1, _KPAD), f32)
    labelrow_ref[...] = jnp.zeros((1, _KPAD), jnp.int32)
    ridrow_ref[...] = jnp.zeros((1, _KPAD), jnp.int32)

    def extract(i, carry):
        p = prob_ref[...]
        m = jnp.max(p)
        fidx = jnp.min(jnp.where(p == m, flat2d, _BIG))
        prob_ref[...] = jnp.where(flat2d == fidx, -1.0, p)

        sel = lane_k == i
        scorerow_ref[...] = jnp.where(sel, m, scorerow_ref[...])
        labelrow_ref[...] = jnp.where(
            sel, jnp.bitwise_and(fidx, _CPAD - 1), labelrow_ref[...])
        ridrow_ref[...] = jnp.where(
            sel, jax.lax.shift_right_logical(fidx, 7), ridrow_ref[...])
        return carry

    jax.lax.fori_loop(0, _K, extract, 0)

    rid_c = jnp.transpose(ridrow_ref[...])
    onehot = (rid_c == jax.lax.broadcasted_iota(
        jnp.int32, (1, _QPAD), 1)).astype(f32)
    bsel = jax.lax.dot_general(
        onehot, bxt_ref[...], (((1,), (0,)), ((), ())),
        precision=jax.lax.Precision.HIGHEST,
        preferred_element_type=f32)
    x0c = bsel[:, 0:1]
    y0c = bsel[:, 1:2]
    x1c = bsel[:, 2:3]
    y1c = bsel[:, 3:4]
    x0r = jnp.transpose(x0c)
    y0r = jnp.transpose(y0c)
    x1r = jnp.transpose(x1c)
    y1r = jnp.transpose(y1c)
    area_r = jnp.maximum(x1r - x0r, 0.0) * jnp.maximum(y1r - y0r, 0.0)
    area_c = jnp.maximum(x1c - x0c, 0.0) * jnp.maximum(y1c - y0c, 0.0)
    iw = jnp.maximum(jnp.minimum(x1c, x1r) - jnp.maximum(x0c, x0r), 0.0)
    ih = jnp.maximum(jnp.minimum(y1c, y1r) - jnp.maximum(y0c, y0r), 0.0)
    inter = iw * ih
    union = area_c + area_r - inter
    iou_ref[...] = inter / jnp.maximum(union, 1e-9)

    keeprow_ref[...] = jnp.ones((1, _KPAD), f32)

    def nms(i, carry):
        krow = keeprow_ref[...]
        ki = jnp.max(jnp.where(lane_k == i, krow, 0.0))
        iourow = iou_ref[pl.ds(i, 1), :]
        sup = (iourow > _IOU_T) & (lane_k > i) & (ki > 0.5)
        keeprow_ref[...] = jnp.where(sup, 0.0, krow)
        return carry

    jax.lax.fori_loop(0, _K, nms, 0)

    scores_ref[0] = scorerow_ref[...]
    labels_ref[0] = labelrow_ref[...]
    keep_ref[0] = keeprow_ref[...]
    boxes_ref[0, :, 0:1] = x0c
    boxes_ref[0, :, 1:2] = y0c
    boxes_ref[0, :, 2:3] = x1c
    boxes_ref[0, :, 3:4] = y1c


@functools.partial(jax.jit)
def _run(lsl, pm, pb, scale):
    f32 = jnp.float32
    grid = (lsl.shape[0],)
    out = pl.pallas_call(
        _body,
        grid=grid,
        in_specs=[
            pl.BlockSpec((1, _Q, _TOK), lambda b: (b, 0, 0)),
            pl.BlockSpec((_C, _TOK), lambda b: (0, 0)),
            pl.BlockSpec((1, _Q, 4), lambda b: (b, 0, 0)),
            pl.BlockSpec((1, 1, 4), lambda b: (b, 0, 0)),
        ],
        out_specs=[
            pl.BlockSpec((1, 1, _KPAD), lambda b: (b, 0, 0)),
            pl.BlockSpec((1, 1, _KPAD), lambda b: (b, 0, 0)),
            pl.BlockSpec((1, _KPAD, 4), lambda b: (b, 0, 0)),
            pl.BlockSpec((1, 1, _KPAD), lambda b: (b, 0, 0)),
        ],
        out_shape=[
            jax.ShapeDtypeStruct((grid[0], 1, _KPAD), f32),
            jax.ShapeDtypeStruct((grid[0], 1, _KPAD), jnp.int32),
            jax.ShapeDtypeStruct((grid[0], _KPAD, 4), f32),
            jax.ShapeDtypeStruct((grid[0], 1, _KPAD), f32),
        ],
        scratch_shapes=[
            pltpu.VMEM((_QPAD, _CPAD), f32),
            pltpu.VMEM((_QPAD, 4), f32),
            pltpu.VMEM((1, _KPAD), f32),
            pltpu.VMEM((1, _KPAD), jnp.int32),
            pltpu.VMEM((1, _KPAD), jnp.int32),
            pltpu.VMEM((1, _KPAD), f32),
            pltpu.VMEM((_KPAD, _KPAD), f32),
        ],
        compiler_params=pltpu.CompilerParams(
            dimension_semantics=("parallel",)),
    )(lsl, pm, pb, scale)
    return out


def kernel(pred_logits, pred_boxes, pos_map, target_sizes):
    lsl = pred_logits[:, :, :_TOK]
    pm = pos_map[:, :_TOK]
    img_h = target_sizes[:, 0].astype(jnp.float32)
    img_w = target_sizes[:, 1].astype(jnp.float32)
    scale = jnp.stack([img_w, img_h, img_w, img_h], axis=1)[:, None, :]
    s, l, bx, k = _run(lsl, pm, pred_boxes, scale)
    scores = s[:, 0, :_K]
    labels = l[:, 0, :_K]
    boxes = bx[:, :_K, :]
    keep = k[:, 0, :_K] > 0.5
    return scores, labels, boxes, keep
